# Initial kernel scaffold; baseline (speedup 1.0000x reference)
import jax
import jax.numpy as jnp
from jax.experimental import pallas as pl


def kernel(x, Ws0, bs0, Ws1, bs1, Wp0, bp0, Wp1, bp1):
    raise NotImplementedError("write your pallas kernel here")


# speedup vs baseline: 1.3485x; 1.3485x over previous
import jax
import jax.numpy as jnp
from jax.experimental import pallas as pl
from jax.experimental.pallas import tpu as pltpu

_M = 512
_DS = 256
_BT = 2048


def _selector_kernel(x_ref, Ws0_ref, bs0_ref, Ws1_ref, bs1_ref, w_ref):
    t = pl.program_id(0)
    nt = pl.num_programs(0)

    x = x_ref[...]
    h = jnp.maximum(
        jnp.dot(x, Ws0_ref[...], preferred_element_type=jnp.float32) + bs0_ref[...],
        0.0)
    s = jnp.dot(h, Ws1_ref[...], preferred_element_type=jnp.float32) + bs1_ref[...]
    neg = -s
    m = jnp.max(neg, axis=1, keepdims=True)
    e = jnp.exp(neg - m)
    p = e / jnp.sum(e, axis=1, keepdims=True)
    part = jnp.sum(p, axis=0, keepdims=True)

    @pl.when(t == 0)
    def _():
        w_ref[...] = part

    @pl.when(t != 0)
    def _():
        w_ref[...] = w_ref[...] + part

    @pl.when(t == nt - 1)
    def _():
        w = w_ref[...]
        wr = jnp.broadcast_to(w, (_M, _M))
        wc = wr.T
        i_idx = jax.lax.broadcasted_iota(jnp.int32, (_M, _M), 0)
        j_idx = jax.lax.broadcasted_iota(jnp.int32, (_M, _M), 1)
        gt = (wc > wr).astype(jnp.float32)
        tie = jnp.logical_and(wc == wr, i_idx < j_idx).astype(jnp.float32)
        rank = jnp.sum(gt + tie, axis=0, keepdims=True)
        keep = rank < float(_DS)
        wk = jnp.where(keep, w, 0.0)
        w_ref[...] = wk / jnp.sum(wk)


def _predictor_kernel(x_ref, w_ref, Wp0_ref, bp0_ref, Wp1_ref, bp1_ref, out_ref):
    xw = x_ref[...] * w_ref[...]
    h = jnp.maximum(
        jnp.dot(xw, Wp0_ref[...], preferred_element_type=jnp.float32) + bp0_ref[...],
        0.0)
    out_ref[...] = (
        jnp.dot(h, Wp1_ref[...], preferred_element_type=jnp.float32) + bp1_ref[...])


def kernel(x, Ws0, bs0, Ws1, bs1, Wp0, bp0, Wp1, bp1):
    B, M = x.shape
    H = Ws0.shape[1]
    nt = B // _BT

    bs0_2d = bs0.reshape(1, H)
    bs1_2d = bs1.reshape(1, M)
    bp0_2d = bp0.reshape(1, H)
    bp1_2d = bp1.reshape(1, M)

    w = pl.pallas_call(
        _selector_kernel,
        grid=(nt,),
        in_specs=[
            pl.BlockSpec((_BT, M), lambda t: (t, 0)),
            pl.BlockSpec((M, H), lambda t: (0, 0)),
            pl.BlockSpec((1, H), lambda t: (0, 0)),
            pl.BlockSpec((H, M), lambda t: (0, 0)),
            pl.BlockSpec((1, M), lambda t: (0, 0)),
        ],
        out_specs=pl.BlockSpec((1, M), lambda t: (0, 0)),
        out_shape=jax.ShapeDtypeStruct((1, M), jnp.float32),
        compiler_params=pltpu.CompilerParams(
            dimension_semantics=("arbitrary",)),
    )(x, Ws0, bs0_2d, Ws1, bs1_2d)

    out = pl.pallas_call(
        _predictor_kernel,
        grid=(nt,),
        in_specs=[
            pl.BlockSpec((_BT, M), lambda t: (t, 0)),
            pl.BlockSpec((1, M), lambda t: (0, 0)),
            pl.BlockSpec((M, H), lambda t: (0, 0)),
            pl.BlockSpec((1, H), lambda t: (0, 0)),
            pl.BlockSpec((H, M), lambda t: (0, 0)),
            pl.BlockSpec((1, M), lambda t: (0, 0)),
        ],
        out_specs=pl.BlockSpec((_BT, M), lambda t: (t, 0)),
        out_shape=jax.ShapeDtypeStruct((B, M), jnp.float32),
        compiler_params=pltpu.CompilerParams(
            dimension_semantics=("arbitrary",)),
    )(x, w, Wp0, bp0_2d, Wp1, bp1_2d)

    return out
